# baseline (speedup 1.0000x reference)
import functools

import jax
import jax.numpy as jnp
from jax import lax
from jax.experimental import pallas as pl
from jax.experimental.pallas import tpu as pltpu
from jax.experimental.pallas import tpu_sc as plsc

_DIM = 64
_NC = 2
_NS = 16
_NW = _NC * _NS
_NBUF = 2


def _make_gather(n_rows: int, row_len: int):
    assert n_rows % _NW == 0
    rows_per_w = n_rows // _NW
    per_w = rows_per_w * row_len
    assert rows_per_w % _NBUF == 0 and row_len % 8 == 0
    n_groups = rows_per_w // _NBUF

    mesh = plsc.VectorSubcoreMesh(
        core_axis_name="c", subcore_axis_name="s",
        num_cores=_NC, num_subcores=_NS,
    )

    @functools.partial(
        pl.kernel,
        mesh=mesh,
        out_type=jax.ShapeDtypeStruct((n_rows, row_len, 128), jnp.float32),
        scratch_types=[
            pltpu.VMEM((rows_per_w, row_len), jnp.int32),
            [pltpu.VMEM((row_len, 128), jnp.float32) for _ in range(_NBUF)],
            [pltpu.SemaphoreType.DMA for _ in range(_NBUF)],
            [pltpu.SemaphoreType.DMA for _ in range(_NBUF)],
        ],
        compiler_params=pltpu.CompilerParams(use_tc_tiling_on_sc=False),
    )
    def gather(idx_hbm, table_hbm, out_hbm, idx_v, rows, gsem, wsem):
        wid = lax.axis_index("s") * _NC + lax.axis_index("c")
        row0 = wid * rows_per_w

        pltpu.sync_copy(idx_hbm.at[pl.ds(wid * rows_per_w, rows_per_w), :],
                        idx_v)

        def start_gather(r, b):
            pltpu.async_copy(table_hbm.at[idx_v.at[r]], rows[b], gsem[b])

        def wait_gather(b):
            pltpu.make_async_copy(
                table_hbm.at[idx_v.at[0]], rows[b], gsem[b]).wait()

        def start_wb(r, b):
            pltpu.async_copy(rows[b], out_hbm.at[row0 + r], wsem[b])

        def wait_wb(b):
            pltpu.make_async_copy(rows[b], out_hbm.at[0], wsem[b]).wait()

        for b in range(_NBUF):
            start_gather(b, b)

        def body(j, carry):
            a = j * _NBUF
            for b in range(_NBUF):
                wait_gather(b)
                start_wb(a + b, b)
            for b in range(_NBUF):
                wait_wb(b)
                start_gather(a + _NBUF + b, b)
            return carry

        lax.fori_loop(0, n_groups - 1, body, 0)

        a = (n_groups - 1) * _NBUF
        for b in range(_NBUF):
            wait_gather(b)
            start_wb(a + b, b)
        for b in range(_NBUF):
            wait_wb(b)

    return gather


def kernel(token_ids, embedding):
    b, s = token_ids.shape
    n_emb = embedding.shape[0]
    table128 = jnp.broadcast_to(
        embedding[:, None, :], (n_emb, 2, _DIM)).reshape(n_emb, 128)
    out = _make_gather(b, s)(token_ids.astype(jnp.int32), table128)
    return out[:, :, :_DIM]


# speedup vs baseline: 1.1836x; 1.1836x over previous
import functools

import jax
import jax.numpy as jnp
from jax import lax
from jax.experimental import pallas as pl
from jax.experimental.pallas import tpu as pltpu
from jax.experimental.pallas import tpu_sc as plsc

_DIM = 64
_NC = 2
_NS = 16
_NW = _NC * _NS
_NBUF = 4


def _make_gather(n_rows: int, row_len: int):
    assert n_rows % _NW == 0
    rows_per_w = n_rows // _NW
    per_w = rows_per_w * row_len
    assert rows_per_w % _NBUF == 0 and row_len % 8 == 0
    n_groups = rows_per_w // _NBUF

    mesh = plsc.VectorSubcoreMesh(
        core_axis_name="c", subcore_axis_name="s",
        num_cores=_NC, num_subcores=_NS,
    )

    @functools.partial(
        pl.kernel,
        mesh=mesh,
        out_type=jax.ShapeDtypeStruct((n_rows, row_len, 128), jnp.float32),
        scratch_types=[
            pltpu.VMEM((rows_per_w, row_len), jnp.int32),
            [pltpu.VMEM((row_len, _DIM), jnp.float32) for _ in range(_NBUF)],
            [pltpu.SemaphoreType.DMA for _ in range(_NBUF)],
            [pltpu.SemaphoreType.DMA for _ in range(_NBUF)],
        ],
        compiler_params=pltpu.CompilerParams(use_tc_tiling_on_sc=False),
    )
    def gather(idx_hbm, table_hbm, out_hbm, idx_v, rows, gsem, wsem):
        wid = lax.axis_index("s") * _NC + lax.axis_index("c")
        row0 = wid * rows_per_w

        pltpu.sync_copy(idx_hbm.at[pl.ds(wid * rows_per_w, rows_per_w), :],
                        idx_v)

        def start_gather(r, b):
            pltpu.async_copy(table_hbm.at[idx_v.at[r]], rows[b], gsem[b])

        def wait_gather(b):
            pltpu.make_async_copy(
                table_hbm.at[idx_v.at[0]], rows[b], gsem[b]).wait()

        def start_wb(r, b):
            pltpu.async_copy(rows[b], out_hbm.at[row0 + r, :, pl.ds(0, _DIM)], wsem[b])

        def wait_wb(b):
            pltpu.make_async_copy(rows[b], out_hbm.at[0, :, pl.ds(0, _DIM)], wsem[b]).wait()

        for b in range(_NBUF):
            start_gather(b, b)

        def body(j, carry):
            a = j * _NBUF
            for b in range(_NBUF):
                wait_gather(b)
                start_wb(a + b, b)
            for b in range(_NBUF):
                wait_wb(b)
                start_gather(a + _NBUF + b, b)
            return carry

        lax.fori_loop(0, n_groups - 1, body, 0)

        a = (n_groups - 1) * _NBUF
        for b in range(_NBUF):
            wait_gather(b)
            start_wb(a + b, b)
        for b in range(_NBUF):
            wait_wb(b)

    return gather


def kernel(token_ids, embedding):
    b, s = token_ids.shape
    out = _make_gather(b, s)(token_ids.astype(jnp.int32), embedding)
    return out[:, :, :_DIM]
